# baseline (speedup 1.0000x reference)
import functools

import jax
import jax.numpy as jnp
from jax import lax
from jax.experimental import pallas as pl
from jax.experimental.pallas import tpu as pltpu
from jax.experimental.pallas import tpu_sc as plsc

VOCAB = 1000
BATCH = 4096
SEQ = 20

_INFO = plsc.get_sparse_core_info()
_NC = _INFO.num_cores
_NS = _INFO.num_subcores
_NW = _NC * _NS

_D = VOCAB
_DP = 1024
_SP = 24
_BPW = BATCH // _NW
_DEPTH = 2
_MB = 2
_C = _SP * _MB
_G = _BPW // _MB
_NGROUP = _G // _DEPTH


def _make_kernel():
    mesh = plsc.VectorSubcoreMesh(core_axis_name="c", subcore_axis_name="s")

    @functools.partial(
        pl.kernel,
        mesh=mesh,
        out_type=jax.ShapeDtypeStruct((BATCH // _MB, _C, _DP), jnp.float32),
        scratch_types=(
            [pltpu.VMEM((_G, _C), jnp.int32)]
            + [pltpu.VMEM((1, _C, _DP), jnp.float32)
               for _ in range(_DEPTH)]
            + [pltpu.SemaphoreType.DMA for _ in range(2 * _DEPTH)]
        ),
    )
    def body(x_hbm, table_hbm, out_hbm, idx_v, *rest):
        bufs = rest[:_DEPTH]
        gsems = rest[_DEPTH:2 * _DEPTH]
        ssems = rest[2 * _DEPTH:]
        wid = lax.axis_index("s") * _NC + lax.axis_index("c")
        base = wid * _G
        pltpu.sync_copy(x_hbm.at[wid], idx_v)

        def fire_gather(g, j):
            pltpu.async_copy(table_hbm.at[idx_v.at[g]], bufs[j].at[0],
                             gsems[j])

        def wait_gather(g, j):
            pltpu.make_async_copy(
                table_hbm.at[idx_v.at[g]], bufs[j].at[0], gsems[j]).wait()

        def _scatter_args(g, j):
            src = bufs[j]
            dst = out_hbm.at[pl.ds(base + g, 1)]
            return src, dst

        def fire_scatter(g, j):
            src, dst = _scatter_args(g, j)
            pltpu.async_copy(src, dst, ssems[j])

        def wait_scatter(g, j):
            src, dst = _scatter_args(g, j)
            pltpu.make_async_copy(src, dst, ssems[j]).wait()

        for j in range(_DEPTH):
            fire_gather(j, j)

        def group(gg, carry):
            for j in range(_DEPTH):
                g = gg * _DEPTH + j
                wait_gather(g, j)
                fire_scatter(g, j)
            for j in range(_DEPTH):
                g = gg * _DEPTH + j
                wait_scatter(g, j)
                fire_gather(g + _DEPTH, j)
            return carry

        lax.fori_loop(0, _NGROUP - 1, group, 0)

        for j in range(_DEPTH):
            g = (_NGROUP - 1) * _DEPTH + j
            wait_gather(g, j)
            fire_scatter(g, j)
        for j in range(_DEPTH):
            g = (_NGROUP - 1) * _DEPTH + j
            wait_scatter(g, j)

    return body


_kernel_call = _make_kernel()


def kernel(x, embedding):
    xr = x.astype(jnp.int32).reshape(_NW, _BPW, SEQ)
    idx = jnp.concatenate([xr, xr[:, :, : _SP - SEQ]], axis=-1)
    idx = idx.reshape(_NW, _G, _C)
    table = jnp.pad(embedding, ((0, 0), (0, _DP - _D)))
    out = _kernel_call(idx, table)
    out = out.reshape(BATCH, _SP, _DP)
    return out[:, :SEQ, :_D]
